# baseline (speedup 1.0000x reference)
import functools

import jax
import jax.numpy as jnp
from jax.experimental import pallas as pl

NUM_CENTROIDS_K = 1024
COMMIT = 0.25
BLOCK_M = 1024


def _vq_block(x_ref, cb_ref, q_ref, loss_ref, idx_ref):
    x = x_ref[...]
    cb = cb_ref[...]
    xsq = jnp.sum(x * x, axis=1, keepdims=True)
    csq = jnp.sum(cb * cb, axis=1)[None, :]
    xc = jax.lax.dot_general(
        x, cb, (((1,), (1,)), ((), ())),
        preferred_element_type=jnp.float32)
    d = xsq - 2.0 * xc + csq
    dmin = jnp.min(d, axis=1, keepdims=True)
    cols = jax.lax.broadcasted_iota(jnp.int32, d.shape, 1)
    idx = jnp.min(jnp.where(d == dmin, cols, NUM_CENTROIDS_K),
                  axis=1).astype(jnp.int32)
    idx_ref[0, 0, :] = idx
    oh = (cols == idx[:, None]).astype(jnp.float32)
    q = jax.lax.dot_general(
        oh, cb, (((1,), (0,)), ((), ())),
        preferred_element_type=jnp.float32)
    diff = q - x
    sq = diff * diff
    loss_ref[...] = sq + COMMIT * sq
    q_ref[...] = x + diff


@functools.partial(jax.jit, static_argnames=())
def _vq_call(flat, codebook):
    m, dim = flat.shape
    grid = m // BLOCK_M
    q, loss, idx = pl.pallas_call(
        _vq_block,
        grid=(grid,),
        in_specs=[
            pl.BlockSpec((BLOCK_M, dim), lambda i: (i, 0)),
            pl.BlockSpec((NUM_CENTROIDS_K, dim), lambda i: (0, 0)),
        ],
        out_specs=[
            pl.BlockSpec((BLOCK_M, dim), lambda i: (i, 0)),
            pl.BlockSpec((BLOCK_M, dim), lambda i: (i, 0)),
            pl.BlockSpec((1, 1, BLOCK_M), lambda i: (i, 0, 0)),
        ],
        out_shape=[
            jax.ShapeDtypeStruct((m, dim), jnp.float32),
            jax.ShapeDtypeStruct((m, dim), jnp.float32),
            jax.ShapeDtypeStruct((grid, 1, BLOCK_M), jnp.int32),
        ],
    )(flat, codebook)
    return q, loss, idx


def kernel(inputs, train, codebook, cluster_counts):
    shape = inputs.shape
    dim = shape[-1]
    flat = inputs.reshape(-1, dim)
    q_st, loss, idx = _vq_call(flat, codebook)
    quantized_st = q_st.reshape(shape)
    quantization_loss = loss.reshape(shape)
    nn_idx_out = idx.reshape(shape[:-1])[None, ...]
    codebook_values = jax.lax.stop_gradient(codebook[None, ...])
    return quantized_st, quantization_loss, nn_idx_out, codebook_values, cluster_counts
